# baseline (speedup 1.0000x reference)
import functools

import jax
import jax.numpy as jnp
from jax import lax
from jax.experimental import pallas as pl
from jax.experimental.pallas import tpu as pltpu
from jax.experimental.pallas import tpu_sc as plsc


def _make_sc_gather(V, DP, B, NC, NS):
    NW = NC * NS
    b_per_w = B // NW
    mesh = plsc.VectorSubcoreMesh(core_axis_name="c", subcore_axis_name="s")

    @functools.partial(
        pl.kernel,
        out_type=jax.ShapeDtypeStruct((B, DP), jnp.float32),
        mesh=mesh,
        scratch_types=[
            pltpu.VMEM((b_per_w,), jnp.int32),
            pltpu.VMEM((b_per_w, DP), jnp.float32),
            pltpu.SemaphoreType.DMA,
        ],
        compiler_params=pltpu.CompilerParams(use_tc_tiling_on_sc=True),
    )
    def gather_k(table_hbm, idx_hbm, out_hbm, idx_v, rows_v, sem):
        wid = lax.axis_index("s") * NC + lax.axis_index("c")
        base = wid * b_per_w
        pltpu.sync_copy(idx_hbm.at[pl.ds(base, b_per_w)], idx_v)
        pltpu.async_copy(table_hbm.at[idx_v], rows_v, sem).wait()
        pltpu.sync_copy(rows_v, out_hbm.at[pl.ds(base, b_per_w)])

    return gather_k


def _pad_body(t_ref, o_ref):
    o_ref[:, :64] = t_ref[...]
    o_ref[:, 64:] = jnp.zeros_like(t_ref[...])


def _matmul_body(x_ref, pos_ref, w_ref, out_ref):
    x = x_ref[...][:, :64] + pos_ref[...]
    out_ref[...] = lax.dot_general(
        x, w_ref[...],
        dimension_numbers=(((1,), (1,)), ((), ())),
        preferred_element_type=jnp.float32)


def kernel(in_idx, tok_emb, pos_emb, W_out):
    B, S = in_idx.shape
    V, D = tok_emb.shape
    DP = 128
    flat_idx = in_idx.reshape(B * S).astype(jnp.int32)

    RT = 4000
    tok_pad = pl.pallas_call(
        _pad_body,
        grid=(V // RT,),
        in_specs=[pl.BlockSpec((RT, D), lambda i: (i, 0))],
        out_specs=pl.BlockSpec((RT, DP), lambda i: (i, 0)),
        out_shape=jax.ShapeDtypeStruct((V, DP), jnp.float32),
    )(tok_emb)

    info = plsc.get_sparse_core_info()
    gather = _make_sc_gather(V, DP, B * S, info.num_cores, info.num_subcores)
    xg = gather(tok_pad, flat_idx)

    M = B * S
    VT = 1024
    pos_full = jnp.tile(pos_emb, (B, 1))
    logits = pl.pallas_call(
        _matmul_body,
        grid=(pl.cdiv(V, VT),),
        in_specs=[
            pl.BlockSpec((M, DP), lambda j: (0, 0)),
            pl.BlockSpec((M, D), lambda j: (0, 0)),
            pl.BlockSpec((VT, D), lambda j: (j, 0)),
        ],
        out_specs=pl.BlockSpec((M, VT), lambda j: (0, j)),
        out_shape=jax.ShapeDtypeStruct((M, V), jnp.float32),
        compiler_params=pltpu.CompilerParams(
            dimension_semantics=("arbitrary",)),
    )(xg, pos_full, W_out)
    return logits.reshape(B, S, V)


# speedup vs baseline: 2.8969x; 2.8969x over previous
import functools

import jax
import jax.numpy as jnp
from jax import lax
from jax.experimental import pallas as pl
from jax.experimental.pallas import tpu as pltpu
from jax.experimental.pallas import tpu_sc as plsc


def _make_sc_xt_gather(V, D, S, T, NC, NS):
    NW = NC * NS
    rows_per_w = D // NW
    mesh = plsc.VectorSubcoreMesh(core_axis_name="c", subcore_axis_name="s")

    @functools.partial(
        pl.kernel,
        out_type=jax.ShapeDtypeStruct((D * T,), jnp.float32),
        mesh=mesh,
        scratch_types=[
            pltpu.VMEM((T,), jnp.int32),
            pltpu.VMEM((T,), jnp.int32),
            pltpu.VMEM((T,), jnp.float32),
            pltpu.VMEM((S,), jnp.float32),
            pltpu.SemaphoreType.DMA,
        ],
        compiler_params=pltpu.CompilerParams(use_tc_tiling_on_sc=True),
    )
    def xt_gather(tok_hbm, idx_hbm, pos_hbm, out_hbm,
                  idx_v, gidx_v, vals_v, pos_v, sem):
        wid = lax.axis_index("s") * NC + lax.axis_index("c")
        pltpu.sync_copy(idx_hbm, idx_v)
        for r in range(rows_per_w):
            d = wid * rows_per_w + r

            pltpu.sync_copy(pos_hbm.at[pl.ds(pl.multiple_of(d * S, 8), S)],
                            pos_v)

            def build(k, _):
                sl = pl.ds(pl.multiple_of(k * 16, 16), 16)
                gidx_v[sl] = idx_v[sl] + d * V
                return 0
            lax.fori_loop(0, T // 16, build, 0)

            def fire_drain(g, _):
                for u in range(8):
                    sl = pl.ds(pl.multiple_of((g * 8 + u) * 128, 128), 128)
                    pltpu.make_async_copy(
                        tok_hbm.at[gidx_v.at[sl]], vals_v.at[sl], sem).start()
                for u in range(8):
                    sl = pl.ds(pl.multiple_of((g * 8 + u) * 128, 128), 128)
                    pltpu.make_async_copy(
                        tok_hbm.at[gidx_v.at[sl]], vals_v.at[sl], sem).wait()
                return 0
            lax.fori_loop(0, T // 128 // 8, fire_drain, 0)

            def add_pos(k, _):
                sl = pl.ds(pl.multiple_of(k * 16, 16), 16)
                p = pl.ds(pl.multiple_of(lax.rem(k, S // 16) * 16, 16), 16)
                vals_v[sl] = vals_v[sl] + pos_v[p]
                return 0
            lax.fori_loop(0, T // 16, add_pos, 0)

            pltpu.sync_copy(vals_v,
                            out_hbm.at[pl.ds(pl.multiple_of(d * T, 8), T)])

    return xt_gather


def _matmul_body(w_ref, xt_ref, out_ref):
    out_ref[0] = lax.dot_general(
        w_ref[...], xt_ref[...],
        dimension_numbers=(((0,), (0,)), ((), ())),
        preferred_element_type=jnp.float32)


def kernel(in_idx, tok_emb, pos_emb, W_out):
    B, S = in_idx.shape
    V, D = tok_emb.shape
    T = B * S
    flat_idx = in_idx.reshape(T).astype(jnp.int32)

    tok_flat = tok_emb.T.reshape(D * V)
    pos_flat = pos_emb.T.reshape(D * S)

    info = plsc.get_sparse_core_info()
    gather = _make_sc_xt_gather(V, D, S, T, info.num_cores, info.num_subcores)
    xt = gather(tok_flat, flat_idx, pos_flat).reshape(D, T)

    VT = 4096
    W_t = W_out.T

    logits_t = pl.pallas_call(
        _matmul_body,
        grid=(B, pl.cdiv(V, VT)),
        in_specs=[
            pl.BlockSpec((D, VT), lambda b, j: (0, j)),
            pl.BlockSpec((D, S), lambda b, j: (0, b)),
        ],
        out_specs=pl.BlockSpec((1, VT, S), lambda b, j: (b, j, 0)),
        out_shape=jax.ShapeDtypeStruct((B, V, S), jnp.float32),
        compiler_params=pltpu.CompilerParams(
            dimension_semantics=("parallel", "arbitrary"),
            fuse_transposed_lhs_in_matmul=True),
    )(W_t, xt)
    return logits_t.transpose(0, 2, 1)
